# baseline (speedup 1.0000x reference)
import functools

import jax
import jax.numpy as jnp
from jax import lax
from jax.experimental import pallas as pl
from jax.experimental.pallas import tpu as pltpu
from jax.experimental.pallas import tpu_sc as plsc

N = 10000
NP = 10240
E = 160000
F = 256
Q = 64
C = 64

CHUNK = 64
EP = 163840
NCHUNK = EP // CHUNK
NC, NS = 2, 16
CPS = NCHUNK // NS
CPW = NCHUNK // (NC * NS)
RPT = NP // NS
ZROWS = 128

BN = 1024


def _pack_cols(sq):
    groups = []
    for g in range(2):
        x = sq[:, 32 * g: 32 * g + 16]
        y = sq[:, 32 * g + 16: 32 * g + 32]
        groups.append(jnp.stack([x, y], axis=-1).reshape(sq.shape[0], 32))
    return jnp.concatenate(groups, axis=1).astype(jnp.bfloat16)


def _split4(s, orefs):
    for k in range(4):
        orefs[k][...] = _pack_cols(s[:, k * Q:(k + 1) * Q])


def _mm1_body(x_ref, w_ref, o0, o1, o2, o3):
    s = jnp.dot(x_ref[...], w_ref[...], preferred_element_type=jnp.float32)
    _split4(s, (o0, o1, o2, o3))


_mm1 = pl.pallas_call(
    _mm1_body,
    grid=(NP // BN,),
    in_specs=[
        pl.BlockSpec((BN, F), lambda i: (i, 0)),
        pl.BlockSpec((F, F), lambda i: (0, 0)),
    ],
    out_specs=[pl.BlockSpec((BN, Q), lambda i: (i, 0)) for _ in range(4)],
    out_shape=[jax.ShapeDtypeStruct((NP, Q), jnp.bfloat16) for _ in range(4)],
)


def _mm2_body(g0, g1, g2, g3, b_ref, w_ref, o0, o1, o2, o3):
    h = jnp.concatenate([g0[...], g1[...], g2[...], g3[...]], axis=1)
    h = jnp.maximum(h + b_ref[...], 0.0)
    s = jnp.dot(h, w_ref[...], preferred_element_type=jnp.float32)
    _split4(s, (o0, o1, o2, o3))


_mm2 = pl.pallas_call(
    _mm2_body,
    grid=(NP // BN,),
    in_specs=[pl.BlockSpec((BN, Q), lambda i: (i, 0)) for _ in range(4)] + [
        pl.BlockSpec((1, F), lambda i: (0, 0)),
        pl.BlockSpec((F, F), lambda i: (0, 0)),
    ],
    out_specs=[pl.BlockSpec((BN, Q), lambda i: (i, 0)) for _ in range(4)],
    out_shape=[jax.ShapeDtypeStruct((NP, Q), jnp.bfloat16) for _ in range(4)],
)


def _mm3_body(g0, g1, g2, g3, b_ref, w_ref, o_ref):
    h = jnp.concatenate([g0[...], g1[...], g2[...], g3[...]], axis=1)
    h = jnp.maximum(h + b_ref[...], 0.0)
    o_ref[...] = _pack_cols(
        jnp.dot(h, w_ref[...], preferred_element_type=jnp.float32))


_mm3 = pl.pallas_call(
    _mm3_body,
    grid=(NP // BN,),
    in_specs=[pl.BlockSpec((BN, Q), lambda i: (i, 0)) for _ in range(4)] + [
        pl.BlockSpec((1, F), lambda i: (0, 0)),
        pl.BlockSpec((F, C), lambda i: (0, 0)),
    ],
    out_specs=pl.BlockSpec((BN, C), lambda i: (i, 0)),
    out_shape=jax.ShapeDtypeStruct((NP, C), jnp.bfloat16),
)


def _fin_body(p0_ref, p1_ref, b_ref, o_ref):
    lg = p0_ref[...] + p1_ref[...] + b_ref[...]
    m = jnp.max(lg, axis=1, keepdims=True)
    ex = jnp.exp(lg - m)
    lse = jnp.log(jnp.sum(ex, axis=1, keepdims=True))
    o_ref[...] = lg - m - lse


_fin = pl.pallas_call(
    _fin_body,
    grid=(NP // BN,),
    in_specs=[
        pl.BlockSpec((BN, C), lambda i: (i, 0)),
        pl.BlockSpec((BN, C), lambda i: (i, 0)),
        pl.BlockSpec((1, C), lambda i: (0, 0)),
    ],
    out_specs=pl.BlockSpec((BN, C), lambda i: (i, 0)),
    out_shape=jax.ShapeDtypeStruct((NP, C), jnp.float32),
)


def _zero_acc(zb_v, acc, sid):
    def zrow(i, _):
        for k in range(Q // 16):
            zb_v[i, pl.ds(k * 16, 16)] = jnp.zeros((16,), jnp.float32)
        return 0

    lax.fori_loop(0, ZROWS, zrow, 0)
    for r in range(RPT // ZROWS):
        pltpu.sync_copy(zb_v, acc.at[pl.ds(sid * RPT + r * ZROWS, ZROWS), :])


def _scale_rows(src_v, dst_v, w_all, j):
    def grp_body(g, _):
        wv = w_all[j, pl.ds(g * 16, 16)]
        for t in range(16):
            w = wv[t]
            r = g * 16 + t
            for k in range(Q // 32):
                raw = src_v[r, pl.ds(k * 32, 32)]
                lo, hi = plsc.unpack(raw, format=plsc.PackFormat.INTERLEAVED)
                dst_v[r, pl.ds(k * 32, 16)] = lo * w
                dst_v[r, pl.ds(k * 32 + 16, 16)] = hi * w
        return 0

    lax.fori_loop(0, CHUNK // 16, grp_body, 0)


def _load_shard(srcm, dstm, ewm, idxs_all, idxd_all, w_all, row0, rows):
    pltpu.sync_copy(srcm.at[pl.ds(row0, rows), :], idxs_all)
    pltpu.sync_copy(dstm.at[pl.ds(row0, rows), :], idxd_all)
    pltpu.sync_copy(ewm.at[pl.ds(row0, rows), :], w_all)


NBUF = 4


def _edge_pass(sup_ref, acc, gbufs, sbufs, zb_v, idxs_all, idxd_all, w_all,
               gsems, ssems, nchunks):
    nsteps = nchunks // NBUF

    def gth(j, b):
        pltpu.async_copy(sup_ref.at[idxs_all.at[j]], gbufs[b], gsems[b])

    def wait_gth(j, b):
        pltpu.make_async_copy(sup_ref.at[idxs_all.at[j]], gbufs[b],
                              gsems[b]).wait()

    def sct(j, b):
        pltpu.async_copy(sbufs[b], acc.at[idxd_all.at[j]], ssems[b], add=True)

    def wait_sct(j, b):
        pltpu.make_async_copy(sbufs[b], acc.at[idxd_all.at[j]],
                              ssems[b]).wait()

    for b in range(NBUF):
        gth(b, b)
        pltpu.async_copy(zb_v.at[pl.ds(0, CHUNK), :], acc.at[idxd_all.at[b]],
                         ssems[b], add=True)

    def step(jj, _):
        for b in range(NBUF):
            j = jj * NBUF + b
            wait_gth(j, b)
            wait_sct(jnp.maximum(j - NBUF, 0), b)
            _scale_rows(gbufs[b], sbufs[b], w_all, j)
            sct(j, b)
            gth(jnp.minimum(j + NBUF, nchunks - 1), b)
        return 0

    lax.fori_loop(0, nsteps, step, 0)

    for b in range(NBUF):
        wait_gth(nchunks - 1, b)
        wait_sct(nchunks - NBUF + b, b)


def _readout(acc, out_ref, sid):
    pltpu.sync_copy(acc.at[pl.ds(sid * RPT, RPT), :],
                    out_ref.at[pl.ds(sid * RPT, RPT), :])


def _sc_mesh():
    return plsc.VectorSubcoreMesh(core_axis_name="c", subcore_axis_name="s",
                                  num_cores=NC, num_subcores=NS)


def _sc_scratch(cps):
    return ([
        pltpu.VMEM((cps, CHUNK), jnp.int32),
        pltpu.VMEM((cps, CHUNK), jnp.int32),
        pltpu.VMEM((cps, CHUNK), jnp.float32),
    ]
        + [pltpu.VMEM((CHUNK, Q), jnp.bfloat16) for _ in range(NBUF)]
        + [pltpu.VMEM((CHUNK, Q), jnp.float32) for _ in range(NBUF)]
        + [pltpu.VMEM((ZROWS, Q), jnp.float32),
           pltpu.VMEM_SHARED((NP, Q), jnp.float32)]
        + [pltpu.SemaphoreType.DMA for _ in range(2 * NBUF)])


@functools.cache
def _sc_spmm_quarter():
    @functools.partial(
        pl.kernel,
        out_type=(
            jax.ShapeDtypeStruct((NP, Q), jnp.float32),
            jax.ShapeDtypeStruct((NP, Q), jnp.float32),
        ),
        mesh=_sc_mesh(),
        scratch_types=_sc_scratch(CPS),
        compiler_params=pltpu.CompilerParams(use_tc_tiling_on_sc=False,
                                             needs_layout_passes=False),
    )
    def spmm_q(supL, supR, srcm, dstm, ewm, tok, outL, outR,
               idxs_all, idxd_all, w_all, *rest):
        del tok
        gbufs = rest[:NBUF]
        sbufs = rest[NBUF:2 * NBUF]
        zb_v, acc = rest[2 * NBUF], rest[2 * NBUF + 1]
        gsems = rest[2 * NBUF + 2:3 * NBUF + 2]
        ssems = rest[3 * NBUF + 2:]
        cid = lax.axis_index("c")
        sid = lax.axis_index("s")

        _zero_acc(zb_v, acc, sid)
        _load_shard(srcm, dstm, ewm, idxs_all, idxd_all, w_all,
                    sid * CPS, CPS)
        plsc.subcore_barrier()

        @pl.when(cid == 0)
        def _():
            _edge_pass(supL, acc, gbufs, sbufs, zb_v, idxs_all, idxd_all, w_all,
                       gsems, ssems, CPS)

        @pl.when(cid == 1)
        def _():
            _edge_pass(supR, acc, gbufs, sbufs, zb_v, idxs_all, idxd_all, w_all,
                       gsems, ssems, CPS)

        plsc.subcore_barrier()

        @pl.when(cid == 0)
        def _():
            _readout(acc, outL, sid)

        @pl.when(cid == 1)
        def _():
            _readout(acc, outR, sid)

    return spmm_q


@functools.cache
def _sc_spmm_esplit():
    @functools.partial(
        pl.kernel,
        out_type=(
            jax.ShapeDtypeStruct((NP, C), jnp.float32),
            jax.ShapeDtypeStruct((NP, C), jnp.float32),
        ),
        mesh=_sc_mesh(),
        scratch_types=_sc_scratch(CPW),
        compiler_params=pltpu.CompilerParams(use_tc_tiling_on_sc=False,
                                             needs_layout_passes=False),
    )
    def spmm_e(sup, srcm, dstm, ewm, tok, out0, out1,
               idxs_all, idxd_all, w_all, *rest):
        del tok
        gbufs = rest[:NBUF]
        sbufs = rest[NBUF:2 * NBUF]
        zb_v, acc = rest[2 * NBUF], rest[2 * NBUF + 1]
        gsems = rest[2 * NBUF + 2:3 * NBUF + 2]
        ssems = rest[3 * NBUF + 2:]
        cid = lax.axis_index("c")
        sid = lax.axis_index("s")
        wid = sid * NC + cid

        _zero_acc(zb_v, acc, sid)
        _load_shard(srcm, dstm, ewm, idxs_all, idxd_all, w_all,
                    wid * CPW, CPW)
        plsc.subcore_barrier()

        _edge_pass(sup, acc, gbufs, sbufs, zb_v, idxs_all, idxd_all, w_all,
                   gsems, ssems, CPW)

        plsc.subcore_barrier()

        @pl.when(cid == 0)
        def _():
            _readout(acc, out0, sid)

        @pl.when(cid == 1)
        def _():
            _readout(acc, out1, sid)

    return spmm_e


def kernel(x, edge_index, edge_weight, W1, b1, W2, b2, W3, b3):
    pad = EP - E
    src = jnp.concatenate([edge_index[0], jnp.zeros((pad,), jnp.int32)])
    dst = jnp.concatenate([edge_index[1], jnp.zeros((pad,), jnp.int32)])
    ew = jnp.concatenate([edge_weight, jnp.zeros((pad,), jnp.float32)])
    srcm = src.reshape(NCHUNK, CHUNK)
    dstm = dst.reshape(NCHUNK, CHUNK)
    ewm = ew.reshape(NCHUNK, CHUNK)

    spmm_q = _sc_spmm_quarter()
    spmm_e = _sc_spmm_esplit()

    def spmm256(q0, q1, q2, q3):
        g0, g1 = spmm_q(q0, q1, srcm, dstm, ewm, q0[:8])
        g2, g3 = spmm_q(q2, q3, srcm, dstm, ewm, g0[:8])
        return g0, g1, g2, g3

    xp = jnp.concatenate([x, jnp.zeros((NP - N, F), jnp.float32)])

    s = _mm1(xp, W1)
    g = spmm256(*s)
    s = _mm2(*g, b1.reshape(1, F), W2)
    g = spmm256(*s)
    s3 = _mm3(*g, b2.reshape(1, F), W3)
    p0, p1 = spmm_e(s3, srcm, dstm, ewm, s3[:8])
    return _fin(p0, p1, b3.reshape(1, C))[:N]


# speedup vs baseline: 2.4807x; 2.4807x over previous
import functools

import jax
import jax.numpy as jnp
from jax import lax
from jax.experimental import pallas as pl
from jax.experimental.pallas import tpu as pltpu
from jax.experimental.pallas import tpu_sc as plsc

N = 10000
NP = 10240
E = 160000
F = 256
Q = 64
C = 64

CHUNK = 64
EP = 163840
NCHUNK = EP // CHUNK
NC, NS = 2, 16
CPS = NCHUNK // NS
CPW = NCHUNK // (NC * NS)
RPT = NP // NS
ZROWS = 128

BN = 1024


QW = Q // 2


def _pack_cols(sq):
    words = []
    for g in range(2):
        x = sq[:, 32 * g: 32 * g + 16]
        y = sq[:, 32 * g + 16: 32 * g + 32]
        xb = lax.bitcast_convert_type(x.astype(jnp.bfloat16),
                                      jnp.uint16).astype(jnp.uint32)
        yb = lax.bitcast_convert_type(y.astype(jnp.bfloat16),
                                      jnp.uint16).astype(jnp.uint32)
        words.append((yb << 16) | xb)
    return lax.bitcast_convert_type(jnp.concatenate(words, axis=1), jnp.int32)


def _split4(s, orefs):
    for k in range(4):
        orefs[k][...] = _pack_cols(s[:, k * Q:(k + 1) * Q])


def _mm1_body(x_ref, w_ref, o0, o1, o2, o3):
    s = jnp.dot(x_ref[...], w_ref[...], preferred_element_type=jnp.float32)
    _split4(s, (o0, o1, o2, o3))


_mm1 = pl.pallas_call(
    _mm1_body,
    grid=(NP // BN,),
    in_specs=[
        pl.BlockSpec((BN, F), lambda i: (i, 0)),
        pl.BlockSpec((F, F), lambda i: (0, 0)),
    ],
    out_specs=[pl.BlockSpec((BN, QW), lambda i: (i, 0)) for _ in range(4)],
    out_shape=[jax.ShapeDtypeStruct((NP, QW), jnp.int32) for _ in range(4)],
)


def _mm2_body(g0, g1, g2, g3, b_ref, w_ref, o0, o1, o2, o3):
    h = jnp.concatenate([g0[...], g1[...], g2[...], g3[...]], axis=1)
    h = jnp.maximum(h + b_ref[...], 0.0)
    s = jnp.dot(h, w_ref[...], preferred_element_type=jnp.float32)
    _split4(s, (o0, o1, o2, o3))


_mm2 = pl.pallas_call(
    _mm2_body,
    grid=(NP // BN,),
    in_specs=[pl.BlockSpec((BN, Q), lambda i: (i, 0)) for _ in range(4)] + [
        pl.BlockSpec((1, F), lambda i: (0, 0)),
        pl.BlockSpec((F, F), lambda i: (0, 0)),
    ],
    out_specs=[pl.BlockSpec((BN, QW), lambda i: (i, 0)) for _ in range(4)],
    out_shape=[jax.ShapeDtypeStruct((NP, QW), jnp.int32) for _ in range(4)],
)


def _mm3_body(g0, g1, g2, g3, b_ref, w_ref, o_ref):
    h = jnp.concatenate([g0[...], g1[...], g2[...], g3[...]], axis=1)
    h = jnp.maximum(h + b_ref[...], 0.0)
    o_ref[...] = _pack_cols(
        jnp.dot(h, w_ref[...], preferred_element_type=jnp.float32))


_mm3 = pl.pallas_call(
    _mm3_body,
    grid=(NP // BN,),
    in_specs=[pl.BlockSpec((BN, Q), lambda i: (i, 0)) for _ in range(4)] + [
        pl.BlockSpec((1, F), lambda i: (0, 0)),
        pl.BlockSpec((F, C), lambda i: (0, 0)),
    ],
    out_specs=pl.BlockSpec((BN, C // 2), lambda i: (i, 0)),
    out_shape=jax.ShapeDtypeStruct((NP, C // 2), jnp.int32),
)


def _fin_body(p0_ref, p1_ref, b_ref, o_ref):
    lg = p0_ref[...] + p1_ref[...] + b_ref[...]
    m = jnp.max(lg, axis=1, keepdims=True)
    ex = jnp.exp(lg - m)
    lse = jnp.log(jnp.sum(ex, axis=1, keepdims=True))
    o_ref[...] = lg - m - lse


_fin = pl.pallas_call(
    _fin_body,
    grid=(NP // BN,),
    in_specs=[
        pl.BlockSpec((BN, C), lambda i: (i, 0)),
        pl.BlockSpec((BN, C), lambda i: (i, 0)),
        pl.BlockSpec((1, C), lambda i: (0, 0)),
    ],
    out_specs=pl.BlockSpec((BN, C), lambda i: (i, 0)),
    out_shape=jax.ShapeDtypeStruct((NP, C), jnp.float32),
)


def _zero_acc(zb_v, acc, sid):
    def zrow(i, _):
        for k in range(Q // 16):
            zb_v[i, pl.ds(k * 16, 16)] = jnp.zeros((16,), jnp.float32)
        return 0

    lax.fori_loop(0, ZROWS, zrow, 0)
    for r in range(RPT // ZROWS):
        pltpu.sync_copy(zb_v, acc.at[pl.ds(sid * RPT + r * ZROWS, ZROWS), :])


def _scale_rows(src_v, dst_v, w_all, j):
    def grp_body(g, _):
        wv = w_all[j, pl.ds(g * 16, 16)]
        for t in range(16):
            w = wv[t]
            r = g * 16 + t
            for k in range(QW // 16):
                raw = src_v[r, pl.ds(k * 16, 16)]
                lo = plsc.bitcast(raw << 16, jnp.float32)
                hi = plsc.bitcast(raw & jnp.int32(-65536), jnp.float32)
                dst_v[r, pl.ds(k * 32, 16)] = lo * w
                dst_v[r, pl.ds(k * 32 + 16, 16)] = hi * w
        return 0

    lax.fori_loop(0, CHUNK // 16, grp_body, 0)


def _load_shard(srcm, dstm, ewm, idxs_all, idxd_all, w_all, row0, rows):
    pltpu.sync_copy(srcm.at[pl.ds(row0, rows), :], idxs_all)
    pltpu.sync_copy(dstm.at[pl.ds(row0, rows), :], idxd_all)
    pltpu.sync_copy(ewm.at[pl.ds(row0, rows), :], w_all)


NBUF = 4


def _edge_pass(sup_ref, acc, gbufs, sbufs, zb_v, idxs_all, idxd_all, w_all,
               gsems, ssems, nchunks):
    nsteps = nchunks // NBUF

    def gth(j, b):
        pltpu.async_copy(sup_ref.at[idxs_all.at[j]], gbufs[b], gsems[b])

    def wait_gth(j, b):
        pltpu.make_async_copy(sup_ref.at[idxs_all.at[j]], gbufs[b],
                              gsems[b]).wait()

    def sct(j, b):
        pltpu.async_copy(sbufs[b], acc.at[idxd_all.at[j]], ssems[b], add=True)

    def wait_sct(j, b):
        pltpu.make_async_copy(sbufs[b], acc.at[idxd_all.at[j]],
                              ssems[b]).wait()

    for b in range(NBUF):
        gth(b, b)
        pltpu.async_copy(zb_v.at[pl.ds(0, CHUNK), :], acc.at[idxd_all.at[b]],
                         ssems[b], add=True)

    def step(jj, _):
        for b in range(NBUF):
            j = jj * NBUF + b
            wait_gth(j, b)
            wait_sct(jnp.maximum(j - NBUF, 0), b)
            _scale_rows(gbufs[b], sbufs[b], w_all, j)
            sct(j, b)
            gth(jnp.minimum(j + NBUF, nchunks - 1), b)
        return 0

    lax.fori_loop(0, nsteps, step, 0)

    for b in range(NBUF):
        wait_gth(nchunks - 1, b)
        wait_sct(nchunks - NBUF + b, b)


def _readout(acc, out_ref, sid):
    pltpu.sync_copy(acc.at[pl.ds(sid * RPT, RPT), :],
                    out_ref.at[pl.ds(sid * RPT, RPT), :])


def _sc_mesh():
    return plsc.VectorSubcoreMesh(core_axis_name="c", subcore_axis_name="s",
                                  num_cores=NC, num_subcores=NS)


def _sc_scratch(cps):
    return ([
        pltpu.VMEM((cps, CHUNK), jnp.int32),
        pltpu.VMEM((cps, CHUNK), jnp.int32),
        pltpu.VMEM((cps, CHUNK), jnp.float32),
    ]
        + [pltpu.VMEM((CHUNK, QW), jnp.int32) for _ in range(NBUF)]
        + [pltpu.VMEM((CHUNK, Q), jnp.float32) for _ in range(NBUF)]
        + [pltpu.VMEM((ZROWS, Q), jnp.float32),
           pltpu.VMEM_SHARED((NP, Q), jnp.float32)]
        + [pltpu.SemaphoreType.DMA for _ in range(2 * NBUF)])


@functools.cache
def _sc_spmm_quarter():
    @functools.partial(
        pl.kernel,
        out_type=(
            jax.ShapeDtypeStruct((NP, Q), jnp.float32),
            jax.ShapeDtypeStruct((NP, Q), jnp.float32),
        ),
        mesh=_sc_mesh(),
        scratch_types=_sc_scratch(CPS),
        compiler_params=pltpu.CompilerParams(use_tc_tiling_on_sc=False,
                                             needs_layout_passes=False),
    )
    def spmm_q(supL, supR, srcm, dstm, ewm, tok, outL, outR,
               idxs_all, idxd_all, w_all, *rest):
        del tok
        gbufs = rest[:NBUF]
        sbufs = rest[NBUF:2 * NBUF]
        zb_v, acc = rest[2 * NBUF], rest[2 * NBUF + 1]
        gsems = rest[2 * NBUF + 2:3 * NBUF + 2]
        ssems = rest[3 * NBUF + 2:]
        cid = lax.axis_index("c")
        sid = lax.axis_index("s")

        _zero_acc(zb_v, acc, sid)
        _load_shard(srcm, dstm, ewm, idxs_all, idxd_all, w_all,
                    sid * CPS, CPS)
        plsc.subcore_barrier()

        @pl.when(cid == 0)
        def _():
            _edge_pass(supL, acc, gbufs, sbufs, zb_v, idxs_all, idxd_all, w_all,
                       gsems, ssems, CPS)

        @pl.when(cid == 1)
        def _():
            _edge_pass(supR, acc, gbufs, sbufs, zb_v, idxs_all, idxd_all, w_all,
                       gsems, ssems, CPS)

        plsc.subcore_barrier()

        @pl.when(cid == 0)
        def _():
            _readout(acc, outL, sid)

        @pl.when(cid == 1)
        def _():
            _readout(acc, outR, sid)

    return spmm_q


@functools.cache
def _sc_spmm_esplit():
    @functools.partial(
        pl.kernel,
        out_type=(
            jax.ShapeDtypeStruct((NP, C), jnp.float32),
            jax.ShapeDtypeStruct((NP, C), jnp.float32),
        ),
        mesh=_sc_mesh(),
        scratch_types=_sc_scratch(CPW),
        compiler_params=pltpu.CompilerParams(use_tc_tiling_on_sc=False,
                                             needs_layout_passes=False),
    )
    def spmm_e(sup, srcm, dstm, ewm, tok, out0, out1,
               idxs_all, idxd_all, w_all, *rest):
        del tok
        gbufs = rest[:NBUF]
        sbufs = rest[NBUF:2 * NBUF]
        zb_v, acc = rest[2 * NBUF], rest[2 * NBUF + 1]
        gsems = rest[2 * NBUF + 2:3 * NBUF + 2]
        ssems = rest[3 * NBUF + 2:]
        cid = lax.axis_index("c")
        sid = lax.axis_index("s")
        wid = sid * NC + cid

        _zero_acc(zb_v, acc, sid)
        _load_shard(srcm, dstm, ewm, idxs_all, idxd_all, w_all,
                    wid * CPW, CPW)
        plsc.subcore_barrier()

        _edge_pass(sup, acc, gbufs, sbufs, zb_v, idxs_all, idxd_all, w_all,
                   gsems, ssems, CPW)

        plsc.subcore_barrier()

        @pl.when(cid == 0)
        def _():
            _readout(acc, out0, sid)

        @pl.when(cid == 1)
        def _():
            _readout(acc, out1, sid)

    return spmm_e


def kernel(x, edge_index, edge_weight, W1, b1, W2, b2, W3, b3):
    pad = EP - E
    src = jnp.concatenate([edge_index[0], jnp.zeros((pad,), jnp.int32)])
    dst = jnp.concatenate([edge_index[1], jnp.zeros((pad,), jnp.int32)])
    ew = jnp.concatenate([edge_weight, jnp.zeros((pad,), jnp.float32)])
    srcm = src.reshape(NCHUNK, CHUNK)
    dstm = dst.reshape(NCHUNK, CHUNK)
    ewm = ew.reshape(NCHUNK, CHUNK)

    spmm_q = _sc_spmm_quarter()
    spmm_e = _sc_spmm_esplit()

    def spmm256(q0, q1, q2, q3):
        g0, g1 = spmm_q(q0, q1, srcm, dstm, ewm, q0[:8])
        g2, g3 = spmm_q(q2, q3, srcm, dstm, ewm, g0[:8])
        return g0, g1, g2, g3

    xp = jnp.concatenate([x, jnp.zeros((NP - N, F), jnp.float32)])

    s = _mm1(xp, W1)
    g = spmm256(*s)
    s = _mm2(*g, b1.reshape(1, F), W2)
    g = spmm256(*s)
    s3 = _mm3(*g, b2.reshape(1, F), W3)
    p0, p1 = spmm_e(s3, srcm, dstm, ewm, s3[:8])
    return _fin(p0, p1, b3.reshape(1, C))[:N]
